# baseline (speedup 1.0000x reference)
import functools

import jax
import jax.numpy as jnp
from jax import lax
from jax.experimental import pallas as pl

B, N, F = 1024, 128, 16
KNN = 7
EPS = 1e-3
BB = 64


def _fold_bn_matmul(w, bnp):
    g, b, m, v = bnp
    s = g / jnp.sqrt(v + EPS)
    return w * s[None, :], (b - m * s)[:, None]


def _bcast(w):
    return jnp.broadcast_to(w[None], (BB,) + w.shape)


def _topk_onehot(d, iota_if, dtype):
    b = lax.bitcast_convert_type(d, jnp.int32)
    s = jnp.where(b >= 0, b, b ^ jnp.int32(0x7FFFFFFF))
    km = (s & jnp.int32(-128)) | iota_if
    ohs = []
    for k in range(KNN + 1):
        mval = jnp.min(km, axis=1, keepdims=True)
        oh = km == mval
        if k > 0:
            ohs.append(oh.astype(dtype))
        km = jnp.where(oh, jnp.int32(0x7FFFFFFF), km)
    return jnp.concatenate(ohs, axis=2)


def _edge_conv(d, iota_if, ftsT, wdT, wbT, t1, w2T, t2, w3T, t3, wscT, tsc):
    mdt = wdT.dtype
    G = _topk_onehot(d, iota_if, mdt)
    knnT = lax.dot_general(ftsT.astype(mdt), G,
                           (((2,), (1,)), ((0,), (0,))),
                           preferred_element_type=jnp.float32)
    uT = lax.dot_general(_bcast(wdT), ftsT.astype(mdt),
                         (((2,), (1,)), ((0,), (0,))),
                         preferred_element_type=jnp.float32) + t1
    c1 = uT.shape[1]
    hp = lax.dot_general(_bcast(wbT), knnT.astype(mdt),
                         (((2,), (1,)), ((0,), (0,))),
                         preferred_element_type=jnp.float32)
    h = jax.nn.relu(hp.reshape(BB, c1, KNN, N) + uT[:, :, None, :])
    h = h.reshape(BB, c1, KNN * N)
    h = jax.nn.relu(
        lax.dot_general(_bcast(w2T), h.astype(mdt),
                        (((2,), (1,)), ((0,), (0,))),
                        preferred_element_type=jnp.float32) + t2)
    h = jax.nn.relu(
        lax.dot_general(_bcast(w3T), h.astype(mdt),
                        (((2,), (1,)), ((0,), (0,))),
                        preferred_element_type=jnp.float32) + t3)
    hm = h[:, :, 0:N]
    for k in range(1, KNN):
        hm = hm + h[:, :, k * N:(k + 1) * N]
    hm = hm * jnp.float32(1.0 / KNN)
    scT = lax.dot_general(_bcast(wscT), ftsT.astype(mdt),
                          (((2,), (1,)), ((0,), (0,))),
                          preferred_element_type=jnp.float32) + tsc
    return jax.nn.relu(scT + hm)


def _net_kernel(fT_ref,
                s0, t0,
                wd0, wb0, t10, w20, t20, w30, t30, wsc0, tsc0,
                wd1, wb1, t11, w21, t21, w31, t31, wsc1, tsc1,
                fcw, fcb, ow, ob,
                o_ref):
    fT = fT_ref[...]
    ftsT = fT * s0[...] + t0[...]
    etaR = fT[:, 0:1, :] * jnp.cos(fT[:, 1:2, :])
    phiR = fT[:, 0:1, :] * jnp.sin(fT[:, 1:2, :])
    redR = jnp.sum(fT, axis=1, keepdims=True)
    maskR = (redR != 0.0).astype(jnp.float32)
    cshiftR = 1e9 * (1.0 - maskR)

    iota_i = lax.broadcasted_iota(jnp.int32, (BB, N, N), 1)
    iota_if = iota_i
    eye3 = (iota_i == lax.broadcasted_iota(jnp.int32, (BB, N, N), 2)
            ).astype(jnp.float32)

    def col(rowvec):
        return lax.dot_general(eye3, rowvec, (((2,), (2,)), ((0,), (0,))),
                               preferred_element_type=jnp.float32)

    peR = cshiftR + etaR
    ppR = cshiftR + phiR
    p01T = jnp.concatenate([peR, ppR], axis=1)
    rAR = peR * peR + ppR * ppR
    mm = lax.dot_general(p01T, p01T, (((1,), (1,)), ((0,), (0,))),
                         preferred_element_type=jnp.float32)
    d1 = col(rAR) - 2.0 * mm
    ftsT = _edge_conv(d1, iota_if, ftsT, wd0[...], wb0[...], t10[...],
                      w20[...], t20[...], w30[...], t30[...], wsc0[...],
                      tsc0[...])

    ptsT = cshiftR + ftsT
    rA2R = jnp.sum(ptsT * ptsT, axis=1, keepdims=True)
    mm2 = lax.dot_general(ptsT, ptsT, (((1,), (1,)), ((0,), (0,))),
                          preferred_element_type=jnp.float32)
    d2 = col(rA2R) - 2.0 * mm2
    ftsT = _edge_conv(d2, iota_if, ftsT, wd1[...], wb1[...], t11[...],
                      w21[...], t21[...], w31[...], t31[...], wsc1[...],
                      tsc1[...])

    ftsmT = ftsT * maskR
    poolc = lax.dot_general(ftsmT, _bcast(jnp.full((N, 1), 1.0 / N,
                                                   jnp.float32)),
                            (((2,), (1,)), ((0,), (0,))),
                            preferred_element_type=jnp.float32)
    h = jax.nn.relu(
        lax.dot_general(_bcast(fcw), poolc, (((2,), (1,)), ((0,), (0,))),
                        preferred_element_type=jnp.float32) + fcb[...])
    lg = lax.dot_general(_bcast(ow), h, (((2,), (1,)), ((0,), (0,))),
                         preferred_element_type=jnp.float32) + ob[...]
    lg = lg - jnp.max(lg, axis=1, keepdims=True)
    e = jnp.exp(lg)
    o_ref[...] = e / jnp.sum(e, axis=1, keepdims=True)


def _prep_weights(params):
    g0, b0, m0, v0 = params["bn0"]
    s0 = (g0 / jnp.sqrt(v0 + EPS))[:, None]
    t0 = (b0 - m0 * g0 / jnp.sqrt(v0 + EPS))[:, None]
    ws = [s0, t0]
    for layer in params["layers"]:
        w1, t1 = _fold_bn_matmul(layer["ws"][0], layer["bns"][0])
        c_in = layer["ws"][0].shape[0] // 2
        wt, wb = w1[:c_in], w1[c_in:]
        w2, t2 = _fold_bn_matmul(layer["ws"][1], layer["bns"][1])
        w3, t3 = _fold_bn_matmul(layer["ws"][2], layer["bns"][2])
        wsc, tsc = _fold_bn_matmul(layer["wsc"], layer["bnsc"])
        mdt = jnp.float32 if not ws[2:] else jnp.bfloat16
        ws += [(wt - wb).T.astype(mdt), wb.T.astype(mdt), t1,
               w2.T.astype(mdt), t2, w3.T.astype(mdt), t3,
               wsc.T.astype(mdt), tsc]
    ws += [params["fc_w"].T, params["fc_b"][:, None], params["out_w"].T,
           params["out_b"][:, None]]
    return ws


@functools.partial(jax.jit, static_argnames=("interpret",))
def _run(features, params, interpret=False):
    ws = _prep_weights(params)
    fT = features.transpose(0, 2, 1)

    def const_spec(a):
        nd = a.ndim
        return pl.BlockSpec(a.shape, lambda i, _nd=nd: (0,) * _nd)

    out = pl.pallas_call(
        _net_kernel,
        grid=(B // BB,),
        in_specs=[pl.BlockSpec((BB, F, N), lambda i: (i, 0, 0))]
        + [const_spec(a) for a in ws],
        out_specs=pl.BlockSpec((BB, 5, 1), lambda i: (i, 0, 0)),
        out_shape=jax.ShapeDtypeStruct((B, 5, 1), jnp.float32),
        interpret=interpret,
    )(fT, *ws)
    return out.reshape(B, 5)


def kernel(features, params):
    return _run(features, params)


# speedup vs baseline: 1.6997x; 1.6997x over previous
import functools

import jax
import jax.numpy as jnp
from jax import lax
from jax.experimental import pallas as pl

B, N, F = 1024, 128, 16
KNN = 7
EPS = 1e-3
BB = 64


def _fold_bn_matmul(w, bnp):
    g, b, m, v = bnp
    s = g / jnp.sqrt(v + EPS)
    return w * s[None, :], (b - m * s)[:, None]


def _bcast(w):
    return jnp.broadcast_to(w[None], (BB,) + w.shape)


def _topk_onehot(d, iota_if, dtype):
    b = lax.bitcast_convert_type(d, jnp.int32)
    s = jnp.where(b >= 0, b, b ^ jnp.int32(0x7FFFFFFF))
    km = (s & jnp.int32(-128)) | iota_if
    ohs = []
    for k in range(KNN + 1):
        mval = jnp.min(km, axis=1, keepdims=True)
        oh = km == mval
        if k > 0:
            ohs.append(oh.astype(dtype))
        km = jnp.where(oh, jnp.int32(0x7FFFFFFF), km)
    return jnp.concatenate(ohs, axis=2)


def _edge_conv(d, iota_if, ftsT, wdT, wbT, t1, w2T, t2, w3T, t3, wscT, tsc):
    mdt = wdT.dtype
    G = _topk_onehot(d, iota_if, mdt)
    knnT = lax.dot_general(ftsT.astype(mdt), G,
                           (((2,), (1,)), ((0,), (0,))),
                           preferred_element_type=jnp.float32)
    uT = lax.dot_general(_bcast(wdT), ftsT.astype(mdt),
                         (((2,), (1,)), ((0,), (0,))),
                         preferred_element_type=jnp.float32) + t1
    utT = jnp.concatenate([uT] * KNN, axis=2)
    h = jax.nn.relu(
        utT
        + lax.dot_general(_bcast(wbT), knnT.astype(mdt),
                          (((2,), (1,)), ((0,), (0,))),
                          preferred_element_type=jnp.float32))
    h = jax.nn.relu(
        lax.dot_general(_bcast(w2T), h.astype(mdt),
                        (((2,), (1,)), ((0,), (0,))),
                        preferred_element_type=jnp.float32) + t2)
    h = jax.nn.relu(
        lax.dot_general(_bcast(w3T), h.astype(mdt),
                        (((2,), (1,)), ((0,), (0,))),
                        preferred_element_type=jnp.float32) + t3)
    hm = h[:, :, 0:N]
    for k in range(1, KNN):
        hm = hm + h[:, :, k * N:(k + 1) * N]
    hm = hm * jnp.float32(1.0 / KNN)
    scT = lax.dot_general(_bcast(wscT), ftsT.astype(mdt),
                          (((2,), (1,)), ((0,), (0,))),
                          preferred_element_type=jnp.float32) + tsc
    return jax.nn.relu(scT + hm)


def _net_kernel(fT_ref,
                s0, t0,
                wd0, wb0, t10, w20, t20, w30, t30, wsc0, tsc0,
                wd1, wb1, t11, w21, t21, w31, t31, wsc1, tsc1,
                fcw, fcb, ow, ob,
                o_ref):
    fT = fT_ref[...]
    ftsT = fT * s0[...] + t0[...]
    etaR = fT[:, 0:1, :] * jnp.cos(fT[:, 1:2, :])
    phiR = fT[:, 0:1, :] * jnp.sin(fT[:, 1:2, :])
    redR = jnp.sum(fT, axis=1, keepdims=True)
    maskR = (redR != 0.0).astype(jnp.float32)
    cshiftR = 1e9 * (1.0 - maskR)

    iota_i = lax.broadcasted_iota(jnp.int32, (BB, N, N), 1)
    iota_if = iota_i
    eye3 = (iota_i == lax.broadcasted_iota(jnp.int32, (BB, N, N), 2)
            ).astype(jnp.float32)

    def col(rowvec):
        return lax.dot_general(eye3, rowvec, (((2,), (2,)), ((0,), (0,))),
                               preferred_element_type=jnp.float32)

    peR = cshiftR + etaR
    ppR = cshiftR + phiR
    p01T = jnp.concatenate([peR, ppR], axis=1)
    rAR = peR * peR + ppR * ppR
    mm = lax.dot_general(p01T, p01T, (((1,), (1,)), ((0,), (0,))),
                         preferred_element_type=jnp.float32)
    d1 = col(rAR) - 2.0 * mm
    ftsT = _edge_conv(d1, iota_if, ftsT, wd0[...], wb0[...], t10[...],
                      w20[...], t20[...], w30[...], t30[...], wsc0[...],
                      tsc0[...])

    ptsT = cshiftR + ftsT
    rA2R = jnp.sum(ptsT * ptsT, axis=1, keepdims=True)
    mm2 = lax.dot_general(ptsT, ptsT, (((1,), (1,)), ((0,), (0,))),
                          preferred_element_type=jnp.float32)
    d2 = col(rA2R) - 2.0 * mm2
    ftsT = _edge_conv(d2, iota_if, ftsT, wd1[...], wb1[...], t11[...],
                      w21[...], t21[...], w31[...], t31[...], wsc1[...],
                      tsc1[...])

    ftsmT = ftsT * maskR
    poolc = lax.dot_general(ftsmT, _bcast(jnp.full((N, 1), 1.0 / N,
                                                   jnp.float32)),
                            (((2,), (1,)), ((0,), (0,))),
                            preferred_element_type=jnp.float32)
    h = jax.nn.relu(
        lax.dot_general(_bcast(fcw), poolc, (((2,), (1,)), ((0,), (0,))),
                        preferred_element_type=jnp.float32) + fcb[...])
    lg = lax.dot_general(_bcast(ow), h, (((2,), (1,)), ((0,), (0,))),
                         preferred_element_type=jnp.float32) + ob[...]
    lg = lg - jnp.max(lg, axis=1, keepdims=True)
    e = jnp.exp(lg)
    o_ref[...] = e / jnp.sum(e, axis=1, keepdims=True)


def _prep_weights(params):
    g0, b0, m0, v0 = params["bn0"]
    s0 = (g0 / jnp.sqrt(v0 + EPS))[:, None]
    t0 = (b0 - m0 * g0 / jnp.sqrt(v0 + EPS))[:, None]
    ws = [s0, t0]
    for layer in params["layers"]:
        w1, t1 = _fold_bn_matmul(layer["ws"][0], layer["bns"][0])
        c_in = layer["ws"][0].shape[0] // 2
        wt, wb = w1[:c_in], w1[c_in:]
        w2, t2 = _fold_bn_matmul(layer["ws"][1], layer["bns"][1])
        w3, t3 = _fold_bn_matmul(layer["ws"][2], layer["bns"][2])
        wsc, tsc = _fold_bn_matmul(layer["wsc"], layer["bnsc"])
        mdt = jnp.float32 if not ws[2:] else jnp.bfloat16
        ws += [(wt - wb).T.astype(mdt), wb.T.astype(mdt), t1,
               w2.T.astype(mdt), t2, w3.T.astype(mdt), t3,
               wsc.T.astype(mdt), tsc]
    ws += [params["fc_w"].T, params["fc_b"][:, None], params["out_w"].T,
           params["out_b"][:, None]]
    return ws


@functools.partial(jax.jit, static_argnames=("interpret",))
def _run(features, params, interpret=False):
    ws = _prep_weights(params)
    fT = features.transpose(0, 2, 1)

    def const_spec(a):
        nd = a.ndim
        return pl.BlockSpec(a.shape, lambda i, _nd=nd: (0,) * _nd)

    out = pl.pallas_call(
        _net_kernel,
        grid=(B // BB,),
        in_specs=[pl.BlockSpec((BB, F, N), lambda i: (i, 0, 0))]
        + [const_spec(a) for a in ws],
        out_specs=pl.BlockSpec((BB, 5, 1), lambda i: (i, 0, 0)),
        out_shape=jax.ShapeDtypeStruct((B, 5, 1), jnp.float32),
        interpret=interpret,
    )(fT, *ws)
    return out.reshape(B, 5)


def kernel(features, params):
    return _run(features, params)


# speedup vs baseline: 1.8340x; 1.0790x over previous
import functools

import jax
import jax.numpy as jnp
from jax import lax
from jax.experimental import pallas as pl

B, N, F = 1024, 128, 16
KNN = 7
EPS = 1e-3
BB = 64


def _fold_bn_matmul(w, bnp):
    g, b, m, v = bnp
    s = g / jnp.sqrt(v + EPS)
    return w * s[None, :], (b - m * s)[:, None]


def _bcast(w):
    return jnp.broadcast_to(w[None], (BB,) + w.shape)


def _topk_onehot(d, iota_if, dtype):
    bi = lax.bitcast_convert_type(d, jnp.int32)
    idxbits = jnp.where(d < 0.0, jnp.int32(N - 1) - iota_if, iota_if)
    km = lax.bitcast_convert_type((bi & jnp.int32(-128)) | idxbits,
                                  jnp.float32)
    ohs = []
    for k in range(KNN + 1):
        mval = jnp.min(km, axis=1, keepdims=True)
        oh = km == mval
        if k > 0:
            ohs.append(oh.astype(dtype))
        km = jnp.where(oh, jnp.float32(3.4028235e38), km)
    return jnp.concatenate(ohs, axis=2)


def _edge_conv(d, iota_if, ftsT, wdT, wbT, t1, w2T, t2, w3T, t3, wscT, tsc):
    mdt = wdT.dtype
    G = _topk_onehot(d, iota_if, mdt)
    knnT = lax.dot_general(ftsT.astype(mdt), G,
                           (((2,), (1,)), ((0,), (0,))),
                           preferred_element_type=jnp.float32)
    uT = lax.dot_general(_bcast(wdT), ftsT.astype(mdt),
                         (((2,), (1,)), ((0,), (0,))),
                         preferred_element_type=jnp.float32) + t1
    utT = jnp.concatenate([uT] * KNN, axis=2)
    h = jax.nn.relu(
        utT
        + lax.dot_general(_bcast(wbT), knnT.astype(mdt),
                          (((2,), (1,)), ((0,), (0,))),
                          preferred_element_type=jnp.float32))
    h = jax.nn.relu(
        lax.dot_general(_bcast(w2T), h.astype(mdt),
                        (((2,), (1,)), ((0,), (0,))),
                        preferred_element_type=jnp.float32) + t2)
    h = jax.nn.relu(
        lax.dot_general(_bcast(w3T), h.astype(mdt),
                        (((2,), (1,)), ((0,), (0,))),
                        preferred_element_type=jnp.float32) + t3)
    hm = h[:, :, 0:N]
    for k in range(1, KNN):
        hm = hm + h[:, :, k * N:(k + 1) * N]
    hm = hm * jnp.float32(1.0 / KNN)
    scT = lax.dot_general(_bcast(wscT), ftsT.astype(mdt),
                          (((2,), (1,)), ((0,), (0,))),
                          preferred_element_type=jnp.float32) + tsc
    return jax.nn.relu(scT + hm)


def _net_kernel(fT_ref,
                s0, t0,
                wd0, wb0, t10, w20, t20, w30, t30, wsc0, tsc0,
                wd1, wb1, t11, w21, t21, w31, t31, wsc1, tsc1,
                fcw, fcb, ow, ob,
                o_ref):
    fT = fT_ref[...]
    ftsT = fT * s0[...] + t0[...]
    etaR = fT[:, 0:1, :] * jnp.cos(fT[:, 1:2, :])
    phiR = fT[:, 0:1, :] * jnp.sin(fT[:, 1:2, :])
    redR = jnp.sum(fT, axis=1, keepdims=True)
    maskR = (redR != 0.0).astype(jnp.float32)
    cshiftR = 1e9 * (1.0 - maskR)

    iota_i = lax.broadcasted_iota(jnp.int32, (BB, N, N), 1)
    iota_if = iota_i
    eye3 = (iota_i == lax.broadcasted_iota(jnp.int32, (BB, N, N), 2)
            ).astype(jnp.float32)

    def col(rowvec):
        return lax.dot_general(eye3, rowvec, (((2,), (2,)), ((0,), (0,))),
                               preferred_element_type=jnp.float32)

    peR = cshiftR + etaR
    ppR = cshiftR + phiR
    p01T = jnp.concatenate([peR, ppR], axis=1)
    rAR = peR * peR + ppR * ppR
    mm = lax.dot_general(p01T, p01T, (((1,), (1,)), ((0,), (0,))),
                         preferred_element_type=jnp.float32)
    d1 = col(rAR) - 2.0 * mm
    ftsT = _edge_conv(d1, iota_if, ftsT, wd0[...], wb0[...], t10[...],
                      w20[...], t20[...], w30[...], t30[...], wsc0[...],
                      tsc0[...])

    ptsT = cshiftR + ftsT
    rA2R = jnp.sum(ptsT * ptsT, axis=1, keepdims=True)
    mm2 = lax.dot_general(ptsT, ptsT, (((1,), (1,)), ((0,), (0,))),
                          preferred_element_type=jnp.float32)
    d2 = col(rA2R) - 2.0 * mm2
    ftsT = _edge_conv(d2, iota_if, ftsT, wd1[...], wb1[...], t11[...],
                      w21[...], t21[...], w31[...], t31[...], wsc1[...],
                      tsc1[...])

    ftsmT = ftsT * maskR
    poolc = lax.dot_general(ftsmT, _bcast(jnp.full((N, 1), 1.0 / N,
                                                   jnp.float32)),
                            (((2,), (1,)), ((0,), (0,))),
                            preferred_element_type=jnp.float32)
    h = jax.nn.relu(
        lax.dot_general(_bcast(fcw), poolc, (((2,), (1,)), ((0,), (0,))),
                        preferred_element_type=jnp.float32) + fcb[...])
    lg = lax.dot_general(_bcast(ow), h, (((2,), (1,)), ((0,), (0,))),
                         preferred_element_type=jnp.float32) + ob[...]
    lg = lg - jnp.max(lg, axis=1, keepdims=True)
    e = jnp.exp(lg)
    o_ref[...] = e / jnp.sum(e, axis=1, keepdims=True)


def _prep_weights(params):
    g0, b0, m0, v0 = params["bn0"]
    s0 = (g0 / jnp.sqrt(v0 + EPS))[:, None]
    t0 = (b0 - m0 * g0 / jnp.sqrt(v0 + EPS))[:, None]
    ws = [s0, t0]
    for layer in params["layers"]:
        w1, t1 = _fold_bn_matmul(layer["ws"][0], layer["bns"][0])
        c_in = layer["ws"][0].shape[0] // 2
        wt, wb = w1[:c_in], w1[c_in:]
        w2, t2 = _fold_bn_matmul(layer["ws"][1], layer["bns"][1])
        w3, t3 = _fold_bn_matmul(layer["ws"][2], layer["bns"][2])
        wsc, tsc = _fold_bn_matmul(layer["wsc"], layer["bnsc"])
        mdt = jnp.float32 if not ws[2:] else jnp.bfloat16
        ws += [(wt - wb).T.astype(mdt), wb.T.astype(mdt), t1,
               w2.T.astype(mdt), t2, w3.T.astype(mdt), t3,
               wsc.T.astype(mdt), tsc]
    ws += [params["fc_w"].T, params["fc_b"][:, None], params["out_w"].T,
           params["out_b"][:, None]]
    return ws


@functools.partial(jax.jit, static_argnames=("interpret",))
def _run(features, params, interpret=False):
    ws = _prep_weights(params)
    fT = features.transpose(0, 2, 1)

    def const_spec(a):
        nd = a.ndim
        return pl.BlockSpec(a.shape, lambda i, _nd=nd: (0,) * _nd)

    out = pl.pallas_call(
        _net_kernel,
        grid=(B // BB,),
        in_specs=[pl.BlockSpec((BB, F, N), lambda i: (i, 0, 0))]
        + [const_spec(a) for a in ws],
        out_specs=pl.BlockSpec((BB, 5, 1), lambda i: (i, 0, 0)),
        out_shape=jax.ShapeDtypeStruct((B, 5, 1), jnp.float32),
        interpret=interpret,
    )(fT, *ws)
    return out.reshape(B, 5)


def kernel(features, params):
    return _run(features, params)


# speedup vs baseline: 1.8392x; 1.0029x over previous
import functools

import jax
import jax.numpy as jnp
from jax import lax
from jax.experimental import pallas as pl

B, N, F = 1024, 128, 16
KNN = 7
EPS = 1e-3
BB = 64


def _fold_bn_matmul(w, bnp):
    g, b, m, v = bnp
    s = g / jnp.sqrt(v + EPS)
    return w * s[None, :], (b - m * s)[:, None]


def _bcast(w):
    return jnp.broadcast_to(w[None], (BB,) + w.shape)


def _topk_onehot(d, iota_if, dtype):
    bi = lax.bitcast_convert_type(d, jnp.int32)
    idxbits = jnp.where(d < 0.0, jnp.int32(N - 1) - iota_if, iota_if)
    km = lax.bitcast_convert_type((bi & jnp.int32(-128)) | idxbits,
                                  jnp.float32)
    ohs = []
    for k in range(KNN + 1):
        mval = jnp.min(km, axis=1, keepdims=True)
        oh = km == mval
        if k > 0:
            ohs.append(oh.astype(dtype))
        km = jnp.where(oh, jnp.float32(3.4028235e38), km)
    return jnp.concatenate(ohs, axis=2)


def _edge_conv(d, iota_if, ftsT, wdT, wbT, t1, w2T, t2, w3T, t3, wscT, tsc):
    mdt = wdT.dtype
    G = _topk_onehot(d, iota_if, mdt)
    knnT = lax.dot_general(ftsT.astype(mdt), G,
                           (((2,), (1,)), ((0,), (0,))),
                           preferred_element_type=jnp.float32)
    uT = lax.dot_general(_bcast(wdT), ftsT.astype(mdt),
                         (((2,), (1,)), ((0,), (0,))),
                         preferred_element_type=jnp.float32) + t1
    utT = jnp.concatenate([uT] * KNN, axis=2)
    h = jax.nn.relu(
        utT
        + lax.dot_general(_bcast(wbT), knnT.astype(mdt),
                          (((2,), (1,)), ((0,), (0,))),
                          preferred_element_type=jnp.float32))
    h = jax.nn.relu(
        lax.dot_general(_bcast(w2T), h.astype(mdt),
                        (((2,), (1,)), ((0,), (0,))),
                        preferred_element_type=jnp.float32) + t2)
    h = jax.nn.relu(
        lax.dot_general(_bcast(w3T), h.astype(mdt),
                        (((2,), (1,)), ((0,), (0,))),
                        preferred_element_type=jnp.float32) + t3)
    hm = h[:, :, 0:N]
    for k in range(1, KNN):
        hm = hm + h[:, :, k * N:(k + 1) * N]
    hm = hm * jnp.float32(1.0 / KNN)
    scT = lax.dot_general(_bcast(wscT), ftsT.astype(mdt),
                          (((2,), (1,)), ((0,), (0,))),
                          preferred_element_type=jnp.float32) + tsc
    return jax.nn.relu(scT + hm)


def _net_kernel(fT_ref,
                s0, t0,
                wd0, wb0, t10, w20, t20, w30, t30, wsc0, tsc0,
                wd1, wb1, t11, w21, t21, w31, t31, wsc1, tsc1,
                fcw, fcb, ow, ob,
                o_ref):
    fT = fT_ref[...]
    ftsT = fT * s0[...] + t0[...]
    etaR = fT[:, 0:1, :] * jnp.cos(fT[:, 1:2, :])
    phiR = fT[:, 0:1, :] * jnp.sin(fT[:, 1:2, :])
    redR = jnp.sum(fT, axis=1, keepdims=True)
    maskR = (redR != 0.0).astype(jnp.float32)
    cshiftR = 1e9 * (1.0 - maskR)

    iota_i = lax.broadcasted_iota(jnp.int32, (BB, N, N), 1)
    iota_if = iota_i
    eye3 = (iota_i == lax.broadcasted_iota(jnp.int32, (BB, N, N), 2)
            ).astype(jnp.float32)

    def col(rowvec):
        return lax.dot_general(eye3, rowvec, (((2,), (2,)), ((0,), (0,))),
                               preferred_element_type=jnp.float32)

    peR = cshiftR + etaR
    ppR = cshiftR + phiR
    p01T = jnp.concatenate([peR, ppR], axis=1)
    rAR = peR * peR + ppR * ppR
    mm = lax.dot_general(p01T, p01T, (((1,), (1,)), ((0,), (0,))),
                         preferred_element_type=jnp.float32)
    d1 = col(rAR) - 2.0 * mm
    ftsT = _edge_conv(d1, iota_if, ftsT, wd0[...], wb0[...], t10[...],
                      w20[...], t20[...], w30[...], t30[...], wsc0[...],
                      tsc0[...])

    ptsT = cshiftR + ftsT
    rA2R = jnp.sum(ptsT * ptsT, axis=1, keepdims=True)
    mm2 = lax.dot_general(ptsT, ptsT, (((1,), (1,)), ((0,), (0,))),
                          preferred_element_type=jnp.float32)
    d2 = col(rA2R) - 2.0 * mm2
    ftsT = _edge_conv(d2, iota_if, ftsT, wd1[...], wb1[...], t11[...],
                      w21[...], t21[...], w31[...], t31[...], wsc1[...],
                      tsc1[...])

    ftsmT = ftsT * maskR
    poolc = lax.dot_general(ftsmT, _bcast(jnp.full((N, 1), 1.0 / N,
                                                   jnp.float32)),
                            (((2,), (1,)), ((0,), (0,))),
                            preferred_element_type=jnp.float32)
    h = jax.nn.relu(
        lax.dot_general(_bcast(fcw), poolc, (((2,), (1,)), ((0,), (0,))),
                        preferred_element_type=jnp.float32) + fcb[...])
    lg = lax.dot_general(_bcast(ow), h, (((2,), (1,)), ((0,), (0,))),
                         preferred_element_type=jnp.float32) + ob[...]
    lg = lg - jnp.max(lg, axis=1, keepdims=True)
    e = jnp.exp(lg)
    o_ref[...] = e / jnp.sum(e, axis=1, keepdims=True)


def _prep_weights(params):
    g0, b0, m0, v0 = params["bn0"]
    s0 = (g0 / jnp.sqrt(v0 + EPS))[:, None]
    t0 = (b0 - m0 * g0 / jnp.sqrt(v0 + EPS))[:, None]
    ws = [s0, t0]
    for layer in params["layers"]:
        w1, t1 = _fold_bn_matmul(layer["ws"][0], layer["bns"][0])
        c_in = layer["ws"][0].shape[0] // 2
        wt, wb = w1[:c_in], w1[c_in:]
        w2, t2 = _fold_bn_matmul(layer["ws"][1], layer["bns"][1])
        w3, t3 = _fold_bn_matmul(layer["ws"][2], layer["bns"][2])
        wsc, tsc = _fold_bn_matmul(layer["wsc"], layer["bnsc"])
        mdt = jnp.float32
        ws += [(wt - wb).T.astype(mdt), wb.T.astype(mdt), t1,
               w2.T.astype(mdt), t2, w3.T.astype(mdt), t3,
               wsc.T.astype(mdt), tsc]
    ws += [params["fc_w"].T, params["fc_b"][:, None], params["out_w"].T,
           params["out_b"][:, None]]
    return ws


@functools.partial(jax.jit, static_argnames=("interpret",))
def _run(features, params, interpret=False):
    ws = _prep_weights(params)
    fT = features.transpose(0, 2, 1)

    def const_spec(a):
        nd = a.ndim
        return pl.BlockSpec(a.shape, lambda i, _nd=nd: (0,) * _nd)

    out = pl.pallas_call(
        _net_kernel,
        grid=(B // BB,),
        in_specs=[pl.BlockSpec((BB, F, N), lambda i: (i, 0, 0))]
        + [const_spec(a) for a in ws],
        out_specs=pl.BlockSpec((BB, 5, 1), lambda i: (i, 0, 0)),
        out_shape=jax.ShapeDtypeStruct((B, 5, 1), jnp.float32),
        interpret=interpret,
    )(fT, *ws)
    return out.reshape(B, 5)


def kernel(features, params):
    return _run(features, params)


# speedup vs baseline: 1.9026x; 1.0345x over previous
import functools

import jax
import jax.numpy as jnp
from jax import lax
from jax.experimental import pallas as pl

B, N, F = 1024, 128, 16
KNN = 7
EPS = 1e-3
BB = 64


def _fold_bn_matmul(w, bnp):
    g, b, m, v = bnp
    s = g / jnp.sqrt(v + EPS)
    return w * s[None, :], (b - m * s)[:, None]


def _bcast(w):
    return jnp.broadcast_to(w[None], (BB,) + w.shape)


def _topk_onehot(d, iota_if, dtype):
    bi = lax.bitcast_convert_type(d, jnp.int32)
    idxbits = jnp.where(d < 0.0, jnp.int32(N - 1) - iota_if, iota_if)
    km = lax.bitcast_convert_type((bi & jnp.int32(-128)) | idxbits,
                                  jnp.float32)
    ohs = []
    for k in range(KNN + 1):
        mval = jnp.min(km, axis=1, keepdims=True)
        oh = km == mval
        if k > 0:
            ohs.append(oh.astype(dtype))
        km = jnp.where(oh, jnp.float32(3.4028235e38), km)
    return jnp.concatenate(ohs, axis=2)


def _edge_conv(d, iota_if, ftsT, wdT, wbT, t1, w2T, t2, w3T, t3, wscT, tsc):
    mdt = wdT.dtype
    G = _topk_onehot(d, iota_if, mdt)
    knnT = lax.dot_general(ftsT.astype(mdt), G,
                           (((2,), (1,)), ((0,), (0,))),
                           preferred_element_type=jnp.float32)
    uT = lax.dot_general(_bcast(wdT), ftsT.astype(mdt),
                         (((2,), (1,)), ((0,), (0,))),
                         preferred_element_type=jnp.float32) + t1
    utT = jnp.concatenate([uT] * KNN, axis=2)
    h = jax.nn.relu(
        utT
        + lax.dot_general(_bcast(wbT), knnT.astype(mdt),
                          (((2,), (1,)), ((0,), (0,))),
                          preferred_element_type=jnp.float32))
    h = jax.nn.relu(
        lax.dot_general(_bcast(w2T), h.astype(mdt),
                        (((2,), (1,)), ((0,), (0,))),
                        preferred_element_type=jnp.float32) + t2)
    h = jax.nn.relu(
        lax.dot_general(_bcast(w3T), h.astype(mdt),
                        (((2,), (1,)), ((0,), (0,))),
                        preferred_element_type=jnp.float32) + t3)
    hm = h[:, :, 0:N]
    for k in range(1, KNN):
        hm = hm + h[:, :, k * N:(k + 1) * N]
    hm = hm * jnp.float32(1.0 / KNN)
    scT = lax.dot_general(_bcast(wscT), ftsT.astype(mdt),
                          (((2,), (1,)), ((0,), (0,))),
                          preferred_element_type=jnp.float32) + tsc
    return jax.nn.relu(scT + hm)


def _net_kernel(fT_ref,
                s0, t0,
                wd0, wb0, t10, w20, t20, w30, t30, wsc0, tsc0,
                wd1, wb1, t11, w21, t21, w31, t31, wsc1, tsc1,
                fcw, fcb, ow, ob,
                o_ref):
    fT = fT_ref[...]
    ftsT = fT * s0[...] + t0[...]
    e0 = (lax.broadcasted_iota(jnp.int32, (1, F), 1) == 0
          ).astype(jnp.float32)
    e1 = (lax.broadcasted_iota(jnp.int32, (1, F), 1) == 1
          ).astype(jnp.float32)
    ch0 = lax.dot_general(_bcast(e0), fT, (((2,), (1,)), ((0,), (0,))),
                          preferred_element_type=jnp.float32)
    ch1 = lax.dot_general(_bcast(e1), fT, (((2,), (1,)), ((0,), (0,))),
                          preferred_element_type=jnp.float32)
    etaR = ch0 * jnp.cos(ch1)
    phiR = ch0 * jnp.sin(ch1)
    redR = jnp.sum(fT, axis=1, keepdims=True)
    maskR = (redR != 0.0).astype(jnp.float32)
    cshiftR = 1e9 * (1.0 - maskR)

    iota_if = lax.broadcasted_iota(jnp.int32, (BB, N, N), 1)

    def scores(ptsT, rAR):
        lhs = jnp.concatenate([ptsT, rAR], axis=1)
        rhs = jnp.concatenate([jnp.float32(-2.0) * ptsT,
                               jnp.ones_like(rAR)], axis=1)
        return lax.dot_general(lhs, rhs, (((1,), (1,)), ((0,), (0,))),
                               preferred_element_type=jnp.float32)

    peR = cshiftR + etaR
    ppR = cshiftR + phiR
    p01T = jnp.concatenate([peR, ppR], axis=1)
    rAR = peR * peR + ppR * ppR
    d1 = scores(p01T, rAR)
    ftsT = _edge_conv(d1, iota_if, ftsT, wd0[...], wb0[...], t10[...],
                      w20[...], t20[...], w30[...], t30[...], wsc0[...],
                      tsc0[...])

    ptsT = cshiftR + ftsT
    rA2R = jnp.sum(ptsT * ptsT, axis=1, keepdims=True)
    d2 = scores(ptsT, rA2R)
    ftsT = _edge_conv(d2, iota_if, ftsT, wd1[...], wb1[...], t11[...],
                      w21[...], t21[...], w31[...], t31[...], wsc1[...],
                      tsc1[...])

    ftsmT = ftsT * maskR
    poolc = lax.dot_general(ftsmT, _bcast(jnp.full((N, 1), 1.0 / N,
                                                   jnp.float32)),
                            (((2,), (1,)), ((0,), (0,))),
                            preferred_element_type=jnp.float32)
    h = jax.nn.relu(
        lax.dot_general(_bcast(fcw), poolc, (((2,), (1,)), ((0,), (0,))),
                        preferred_element_type=jnp.float32) + fcb[...])
    lg = lax.dot_general(_bcast(ow), h, (((2,), (1,)), ((0,), (0,))),
                         preferred_element_type=jnp.float32) + ob[...]
    lg = lg - jnp.max(lg, axis=1, keepdims=True)
    e = jnp.exp(lg)
    o_ref[...] = e / jnp.sum(e, axis=1, keepdims=True)


def _prep_weights(params):
    g0, b0, m0, v0 = params["bn0"]
    s0 = (g0 / jnp.sqrt(v0 + EPS))[:, None]
    t0 = (b0 - m0 * g0 / jnp.sqrt(v0 + EPS))[:, None]
    ws = [s0, t0]
    for layer in params["layers"]:
        w1, t1 = _fold_bn_matmul(layer["ws"][0], layer["bns"][0])
        c_in = layer["ws"][0].shape[0] // 2
        wt, wb = w1[:c_in], w1[c_in:]
        w2, t2 = _fold_bn_matmul(layer["ws"][1], layer["bns"][1])
        w3, t3 = _fold_bn_matmul(layer["ws"][2], layer["bns"][2])
        wsc, tsc = _fold_bn_matmul(layer["wsc"], layer["bnsc"])
        mdt = jnp.float32
        ws += [(wt - wb).T.astype(mdt), wb.T.astype(mdt), t1,
               w2.T.astype(mdt), t2, w3.T.astype(mdt), t3,
               wsc.T.astype(mdt), tsc]
    ws += [params["fc_w"].T, params["fc_b"][:, None], params["out_w"].T,
           params["out_b"][:, None]]
    return ws


@functools.partial(jax.jit, static_argnames=("interpret",))
def _run(features, params, interpret=False):
    ws = _prep_weights(params)
    fT = features.transpose(0, 2, 1)

    def const_spec(a):
        nd = a.ndim
        return pl.BlockSpec(a.shape, lambda i, _nd=nd: (0,) * _nd)

    out = pl.pallas_call(
        _net_kernel,
        grid=(B // BB,),
        in_specs=[pl.BlockSpec((BB, F, N), lambda i: (i, 0, 0))]
        + [const_spec(a) for a in ws],
        out_specs=pl.BlockSpec((BB, 5, 1), lambda i: (i, 0, 0)),
        out_shape=jax.ShapeDtypeStruct((B, 5, 1), jnp.float32),
        interpret=interpret,
    )(fT, *ws)
    return out.reshape(B, 5)


def kernel(features, params):
    return _run(features, params)


# speedup vs baseline: 1.9035x; 1.0005x over previous
import jax
import jax.numpy as jnp
from jax import lax
from jax.experimental import pallas as pl

B, N, F = 1024, 128, 16
KNN = 7
EPS = 1e-3
BB = 64


def _fold_bn_matmul(w, bnp):
    g, b, m, v = bnp
    s = g / jnp.sqrt(v + EPS)
    return w * s[None, :], (b - m * s)[:, None]


def _bcast(w):
    return jnp.broadcast_to(w[None], (BB,) + w.shape)


def _topk_onehot(d, iota_if, dtype):
    bi = lax.bitcast_convert_type(d, jnp.int32)
    idxbits = jnp.where(d < 0.0, jnp.int32(N - 1) - iota_if, iota_if)
    km = lax.bitcast_convert_type((bi & jnp.int32(-128)) | idxbits,
                                  jnp.float32)
    ohs = []
    for k in range(KNN + 1):
        mval = jnp.min(km, axis=1, keepdims=True)
        oh = km == mval
        if k > 0:
            ohs.append(oh.astype(dtype))
        km = jnp.where(oh, jnp.float32(3.4028235e38), km)
    return jnp.concatenate(ohs, axis=2)


def _edge_conv(d, iota_if, ftsT, wdT, wbT, t1, w2T, t2, w3T, t3, wscT, tsc):
    mdt = wdT.dtype
    G = _topk_onehot(d, iota_if, mdt)
    knnT = lax.dot_general(ftsT.astype(mdt), G,
                           (((2,), (1,)), ((0,), (0,))),
                           preferred_element_type=jnp.float32)
    uT = lax.dot_general(_bcast(wdT), ftsT.astype(mdt),
                         (((2,), (1,)), ((0,), (0,))),
                         preferred_element_type=jnp.float32) + t1
    utT = jnp.concatenate([uT] * KNN, axis=2)
    h = jax.nn.relu(
        utT
        + lax.dot_general(_bcast(wbT), knnT.astype(mdt),
                          (((2,), (1,)), ((0,), (0,))),
                          preferred_element_type=jnp.float32))
    h = jax.nn.relu(
        lax.dot_general(_bcast(w2T), h.astype(mdt),
                        (((2,), (1,)), ((0,), (0,))),
                        preferred_element_type=jnp.float32) + t2)
    h = jax.nn.relu(
        lax.dot_general(_bcast(w3T), h.astype(mdt),
                        (((2,), (1,)), ((0,), (0,))),
                        preferred_element_type=jnp.float32) + t3)
    hm = h[:, :, 0:N]
    for k in range(1, KNN):
        hm = hm + h[:, :, k * N:(k + 1) * N]
    hm = hm * jnp.float32(1.0 / KNN)
    scT = lax.dot_general(_bcast(wscT), ftsT.astype(mdt),
                          (((2,), (1,)), ((0,), (0,))),
                          preferred_element_type=jnp.float32) + tsc
    return jax.nn.relu(scT + hm)


def _net_kernel(fT_ref,
                s0, t0,
                wd0, wb0, t10, w20, t20, w30, t30, wsc0, tsc0,
                wd1, wb1, t11, w21, t21, w31, t31, wsc1, tsc1,
                fcw, fcb, ow, ob,
                o_ref):
    fT = fT_ref[...]
    ftsT = fT * s0[...] + t0[...]
    e0 = (lax.broadcasted_iota(jnp.int32, (1, F), 1) == 0
          ).astype(jnp.float32)
    e1 = (lax.broadcasted_iota(jnp.int32, (1, F), 1) == 1
          ).astype(jnp.float32)
    ch0 = lax.dot_general(_bcast(e0), fT, (((2,), (1,)), ((0,), (0,))),
                          preferred_element_type=jnp.float32)
    ch1 = lax.dot_general(_bcast(e1), fT, (((2,), (1,)), ((0,), (0,))),
                          preferred_element_type=jnp.float32)
    etaR = ch0 * jnp.cos(ch1)
    phiR = ch0 * jnp.sin(ch1)
    redR = jnp.sum(fT, axis=1, keepdims=True)
    maskR = (redR != 0.0).astype(jnp.float32)
    cshiftR = 1e9 * (1.0 - maskR)

    iota_if = lax.broadcasted_iota(jnp.int32, (BB, N, N), 1)

    def scores(ptsT, rAR):
        lhs = jnp.concatenate([ptsT, rAR], axis=1)
        rhs = jnp.concatenate([jnp.float32(-2.0) * ptsT,
                               jnp.ones_like(rAR)], axis=1)
        return lax.dot_general(lhs, rhs, (((1,), (1,)), ((0,), (0,))),
                               preferred_element_type=jnp.float32)

    peR = cshiftR + etaR
    ppR = cshiftR + phiR
    p01T = jnp.concatenate([peR, ppR], axis=1)
    rAR = peR * peR + ppR * ppR
    d1 = scores(p01T, rAR)
    ftsT = _edge_conv(d1, iota_if, ftsT, wd0[...], wb0[...], t10[...],
                      w20[...], t20[...], w30[...], t30[...], wsc0[...],
                      tsc0[...])

    ptsT = cshiftR + ftsT
    rA2R = jnp.sum(ptsT * ptsT, axis=1, keepdims=True)
    d2 = scores(ptsT, rA2R)
    ftsT = _edge_conv(d2, iota_if, ftsT, wd1[...], wb1[...], t11[...],
                      w21[...], t21[...], w31[...], t31[...], wsc1[...],
                      tsc1[...])

    ftsmT = ftsT * maskR
    poolc = lax.dot_general(ftsmT, _bcast(jnp.full((N, 1), 1.0 / N,
                                                   jnp.float32)),
                            (((2,), (1,)), ((0,), (0,))),
                            preferred_element_type=jnp.float32)
    h = jax.nn.relu(
        lax.dot_general(_bcast(fcw), poolc, (((2,), (1,)), ((0,), (0,))),
                        preferred_element_type=jnp.float32) + fcb[...])
    lg = lax.dot_general(_bcast(ow), h, (((2,), (1,)), ((0,), (0,))),
                         preferred_element_type=jnp.float32) + ob[...]
    lg = lg - jnp.max(lg, axis=1, keepdims=True)
    e = jnp.exp(lg)
    o_ref[...] = e / jnp.sum(e, axis=1, keepdims=True)


def _prep_weights(params):
    g0, b0, m0, v0 = params["bn0"]
    s0 = (g0 / jnp.sqrt(v0 + EPS))[:, None]
    t0 = (b0 - m0 * g0 / jnp.sqrt(v0 + EPS))[:, None]
    ws = [s0, t0]
    for layer in params["layers"]:
        w1, t1 = _fold_bn_matmul(layer["ws"][0], layer["bns"][0])
        c_in = layer["ws"][0].shape[0] // 2
        wt, wb = w1[:c_in], w1[c_in:]
        w2, t2 = _fold_bn_matmul(layer["ws"][1], layer["bns"][1])
        w3, t3 = _fold_bn_matmul(layer["ws"][2], layer["bns"][2])
        wsc, tsc = _fold_bn_matmul(layer["wsc"], layer["bnsc"])
        mdt = jnp.float32
        ws += [(wt - wb).T.astype(mdt), wb.T.astype(mdt), t1,
               w2.T.astype(mdt), t2, w3.T.astype(mdt), t3,
               wsc.T.astype(mdt), tsc]
    ws += [params["fc_w"].T, params["fc_b"][:, None], params["out_w"].T,
           params["out_b"][:, None]]
    return ws


@jax.jit
def _run(features, params):
    ws = _prep_weights(params)
    fT = features.transpose(0, 2, 1)

    def const_spec(a):
        nd = a.ndim
        return pl.BlockSpec(a.shape, lambda i, _nd=nd: (0,) * _nd)

    out = pl.pallas_call(
        _net_kernel,
        grid=(B // BB,),
        in_specs=[pl.BlockSpec((BB, F, N), lambda i: (i, 0, 0))]
        + [const_spec(a) for a in ws],
        out_specs=pl.BlockSpec((BB, 5, 1), lambda i: (i, 0, 0)),
        out_shape=jax.ShapeDtypeStruct((B, 5, 1), jnp.float32),
    )(fT, *ws)
    return out.reshape(B, 5)


def kernel(features, params):
    return _run(features, params)
